# Initial kernel scaffold; baseline (speedup 1.0000x reference)
import jax
import jax.numpy as jnp
from jax.experimental import pallas as pl


def kernel(world_pos, prev_world_pos, mesh_pos, node_type, cells, params, is_training):
    raise NotImplementedError("write your pallas kernel here")


# speedup vs baseline: 2.1496x; 2.1496x over previous
import functools

import jax
import jax.numpy as jnp
from jax import lax
from jax.experimental import pallas as pl
from jax.experimental.pallas import tpu as pltpu
from jax.experimental.pallas import tpu_sc as plsc

_F32 = jnp.float32
_NC, _NS = 2, 16
_NW = _NC * _NS
_CH = 128
_NB = 1000
_EB = 1920


def _ln(x, g, b):
    mu = jnp.mean(x, axis=-1, keepdims=True)
    xc = x - mu
    var = jnp.mean(xc * xc, axis=-1, keepdims=True)
    return xc * lax.rsqrt(var + 1e-5) * g + b


def _mlp_tail(h, w2, b2, w3, b3, g, be):
    h = jax.nn.relu(jnp.dot(h, w2, preferred_element_type=_F32) + b2)
    x = jnp.dot(h, w3, preferred_element_type=_F32) + b3
    return _ln(x, g, be)


@functools.lru_cache(maxsize=None)
def _sc_gather2(n, width, epad):
    k_per_w = epad // (_NW * _CH)
    mesh = plsc.VectorSubcoreMesh(core_axis_name="c", subcore_axis_name="s")

    @functools.partial(
        pl.kernel,
        out_type=(jax.ShapeDtypeStruct((epad, width), _F32),
                  jax.ShapeDtypeStruct((epad, width), _F32)),
        mesh=mesh,
        scratch_types=[
            pltpu.VMEM((_CH,), jnp.int32),
            pltpu.VMEM((_CH,), jnp.int32),
            pltpu.VMEM((_CH, width), _F32),
            pltpu.VMEM((_CH, width), _F32),
            pltpu.SemaphoreType.DMA,
            pltpu.SemaphoreType.DMA,
        ],
    )
    def k(ta, tb, ia_h, ib_h, ga_h, gb_h, ia, ib, ra, rb, sa, sb):
        wid = lax.axis_index("s") * _NC + lax.axis_index("c")

        def chunk(j):
            base = (wid * k_per_w + j) * _CH
            pltpu.sync_copy(ia_h.at[pl.ds(base, _CH)], ia)
            pltpu.sync_copy(ib_h.at[pl.ds(base, _CH)], ib)
            ca = pltpu.async_copy(ta.at[ia], ra, sa)
            cb = pltpu.async_copy(tb.at[ib], rb, sb)
            ca.wait()
            cb.wait()
            pltpu.sync_copy(ra, ga_h.at[pl.ds(base, _CH)])
            pltpu.sync_copy(rb, gb_h.at[pl.ds(base, _CH)])

        pl.loop(0, k_per_w)(chunk)

    return k


@functools.lru_cache(maxsize=None)
def _sc_scatter(nrows, epad):
    k_per_w = epad // (_NW * _CH)
    rpt = nrows // _NS
    mesh = plsc.VectorSubcoreMesh(core_axis_name="c", subcore_axis_name="s")

    @functools.partial(
        pl.kernel,
        out_type=(jax.ShapeDtypeStruct((nrows, 128), _F32),
                  jax.ShapeDtypeStruct((nrows, 128), _F32)),
        mesh=mesh,
        scratch_types=[
            pltpu.VMEM((_CH,), jnp.int32),
            pltpu.VMEM((_CH, 128), _F32),
            pltpu.VMEM_SHARED((nrows, 128), _F32),
        ],
    )
    def k(ne_h, idx_h, z_h, o0, o1, idx_v, ne_v, acc):
        c = lax.axis_index("c")
        s = lax.axis_index("s")
        wid = s * _NC + c
        r0 = s * rpt
        pltpu.sync_copy(z_h.at[pl.ds(r0, rpt)], acc.at[pl.ds(r0, rpt)])
        plsc.subcore_barrier()

        def chunk(j):
            base = (wid * k_per_w + j) * _CH
            pltpu.sync_copy(idx_h.at[pl.ds(base, _CH)], idx_v)
            pltpu.sync_copy(ne_h.at[pl.ds(base, _CH)], ne_v)
            pltpu.sync_copy(ne_v, acc.at[idx_v], add=True)

        pl.loop(0, k_per_w)(chunk)
        plsc.subcore_barrier()

        @pl.when(c == 0)
        def _():
            pltpu.sync_copy(acc.at[pl.ds(r0, rpt)], o0.at[pl.ds(r0, rpt)])

        @pl.when(c == 1)
        def _():
            pltpu.sync_copy(acc.at[pl.ds(r0, rpt)], o1.at[pl.ds(r0, rpt)])

    return k


def _row(i):
    return (i, 0)


def _full(i):
    return (0, 0)


def _w(shape):
    return pl.BlockSpec(shape, _full)


_PARALLEL = pltpu.CompilerParams(dimension_semantics=("parallel",))


@functools.lru_cache(maxsize=None)
def _node_enc_call(n):
    def body(wp, pw, nt, w1v, w1t, b1, w2, b2, w3, b3, g, be, ws, wr,
             nl_ref, ps_ref, pr_ref):
        vel = wp[...] - pw[...]
        acc = (vel[:, 0:1] * w1v[0:1, :] + vel[:, 1:2] * w1v[1:2, :]
               + vel[:, 2:3] * w1v[2:3, :]) + b1[...]
        t = nt[...]
        for kk in range(9):
            acc = acc + jnp.where(t == kk, 1.0, 0.0) * w1t[kk:kk + 1, :]
        h = jax.nn.relu(acc)
        nl = _mlp_tail(h, w2[...], b2[...], w3[...], b3[...], g[...], be[...])
        nl_ref[...] = nl
        ps_ref[...] = jnp.dot(nl, ws[...], preferred_element_type=_F32)
        pr_ref[...] = jnp.dot(nl, wr[...], preferred_element_type=_F32)

    return pl.pallas_call(
        body,
        grid=(n // _NB,),
        in_specs=[
            pl.BlockSpec((_NB, 3), _row), pl.BlockSpec((_NB, 3), _row),
            pl.BlockSpec((_NB, 1), _row),
            _w((3, 128)), _w((9, 128)), _w((1, 128)),
            _w((128, 128)), _w((1, 128)), _w((128, 128)), _w((1, 128)),
            _w((1, 128)), _w((1, 128)), _w((128, 128)), _w((128, 128)),
        ],
        out_specs=[pl.BlockSpec((_NB, 128), _row)] * 3,
        out_shape=[jax.ShapeDtypeStruct((n, 128), _F32)] * 3,
        compiler_params=_PARALLEL,
    )


@functools.lru_cache(maxsize=None)
def _edge_enc_call(epad):
    def body(ts, tr, wg, b1, w2, b2, w3, b3, g, be, el_ref):
        d = ts[...] - tr[...]
        sq = d * d
        col = lax.broadcasted_iota(jnp.int32, d.shape, 1)
        nw = jnp.sqrt(jnp.sum(jnp.where(col < 3, sq, 0.0), axis=1,
                              keepdims=True))
        nm = jnp.sqrt(jnp.sum(jnp.where((col >= 3) & (col < 5), sq, 0.0),
                              axis=1, keepdims=True))
        gf = jnp.where(col == 5, nw, jnp.where(col == 6, nm, d))
        h = jax.nn.relu(jnp.dot(gf, wg[...], preferred_element_type=_F32)
                        + b1[...])
        el_ref[...] = _mlp_tail(h, w2[...], b2[...], w3[...], b3[...],
                                g[...], be[...])

    return pl.pallas_call(
        body,
        grid=(epad // _EB,),
        in_specs=[
            pl.BlockSpec((_EB, 128), _row), pl.BlockSpec((_EB, 128), _row),
            _w((128, 128)), _w((1, 128)),
            _w((128, 128)), _w((1, 128)), _w((128, 128)), _w((1, 128)),
            _w((1, 128)), _w((1, 128)),
        ],
        out_specs=pl.BlockSpec((_EB, 128), _row),
        out_shape=jax.ShapeDtypeStruct((epad, 128), _F32),
        compiler_params=_PARALLEL,
    )


@functools.lru_cache(maxsize=None)
def _edge_step_call(epad):
    def body(gs, gr, el, w1e, b1, w2, b2, w3, b3, g, be, ne_ref, el_ref):
        x0 = el[...]
        h = jax.nn.relu(gs[...] + gr[...]
                        + jnp.dot(x0, w1e[...], preferred_element_type=_F32)
                        + b1[...])
        ne = _mlp_tail(h, w2[...], b2[...], w3[...], b3[...], g[...], be[...])
        ne_ref[...] = ne
        el_ref[...] = x0 + ne

    return pl.pallas_call(
        body,
        grid=(epad // _EB,),
        in_specs=[
            pl.BlockSpec((_EB, 128), _row), pl.BlockSpec((_EB, 128), _row),
            pl.BlockSpec((_EB, 128), _row),
            _w((128, 128)), _w((1, 128)),
            _w((128, 128)), _w((1, 128)), _w((128, 128)), _w((1, 128)),
            _w((1, 128)), _w((1, 128)),
        ],
        out_specs=[pl.BlockSpec((_EB, 128), _row)] * 2,
        out_shape=[jax.ShapeDtypeStruct((epad, 128), _F32)] * 2,
        compiler_params=_PARALLEL,
    )


@functools.lru_cache(maxsize=None)
def _node_step_call(n, nrows):
    def body(nl, a0, a1, wna, wnb, b1, w2, b2, w3, b3, g, be, ws, wr,
             nl_ref, ps_ref, pr_ref):
        x0 = nl[...]
        agg = a0[...] + a1[...]
        h = jax.nn.relu(jnp.dot(x0, wna[...], preferred_element_type=_F32)
                        + jnp.dot(agg, wnb[...], preferred_element_type=_F32)
                        + b1[...])
        nn = _mlp_tail(h, w2[...], b2[...], w3[...], b3[...], g[...], be[...])
        x1 = x0 + nn
        nl_ref[...] = x1
        ps_ref[...] = jnp.dot(x1, ws[...], preferred_element_type=_F32)
        pr_ref[...] = jnp.dot(x1, wr[...], preferred_element_type=_F32)

    return pl.pallas_call(
        body,
        grid=(n // _NB,),
        in_specs=[
            pl.BlockSpec((_NB, 128), _row),
            pl.BlockSpec((_NB, 128), _row), pl.BlockSpec((_NB, 128), _row),
            _w((128, 128)), _w((128, 128)), _w((1, 128)),
            _w((128, 128)), _w((1, 128)), _w((128, 128)), _w((1, 128)),
            _w((1, 128)), _w((1, 128)), _w((128, 128)), _w((128, 128)),
        ],
        out_specs=[pl.BlockSpec((_NB, 128), _row)] * 3,
        out_shape=[jax.ShapeDtypeStruct((n, 128), _F32)] * 3,
        compiler_params=_PARALLEL,
    )


@functools.lru_cache(maxsize=None)
def _decoder_call(n):
    def body(nl, w1, b1, w2, b2, w3, b3, out_ref):
        h = jax.nn.relu(jnp.dot(nl[...], w1[...],
                                preferred_element_type=_F32) + b1[...])
        h = jax.nn.relu(jnp.dot(h, w2[...],
                                preferred_element_type=_F32) + b2[...])
        out_ref[...] = jnp.dot(h, w3[...],
                               preferred_element_type=_F32) + b3[...]

    return pl.pallas_call(
        body,
        grid=(n // _NB,),
        in_specs=[
            pl.BlockSpec((_NB, 128), _row),
            _w((128, 128)), _w((1, 128)), _w((128, 128)), _w((1, 128)),
            _w((128, 3)), _w((1, 3)),
        ],
        out_specs=pl.BlockSpec((_NB, 3), _row),
        out_shape=jax.ShapeDtypeStruct((n, 3), _F32),
        compiler_params=_PARALLEL,
    )


def _build_edges(cells, n):
    e = jnp.concatenate([
        cells[:, 0:2],
        cells[:, 1:3],
        jnp.stack([cells[:, 2], cells[:, 0]], axis=1),
    ], axis=0)
    recv = jnp.min(e, axis=1)
    send = jnp.max(e, axis=1)
    packed = send * n + recv
    ps = jnp.sort(packed)
    is_first = jnp.concatenate([
        jnp.ones((1,), dtype=bool), ps[1:] != ps[:-1]], axis=0)
    s_u = ps // n
    r_u = ps % n
    senders = jnp.concatenate([s_u, r_u], axis=0)
    receivers = jnp.concatenate([r_u, s_u], axis=0)
    valid = jnp.concatenate([is_first, is_first], axis=0)
    ragg = jnp.where(valid, receivers, n)
    return senders, receivers, ragg


def _fold_first_layer(w1, b1, mean, std):
    w1p = w1 / std[:, None]
    b1p = b1 - (mean / std) @ w1
    return w1p, b1p


def _r2(b):
    return b.reshape(1, -1)


def kernel(world_pos, prev_world_pos, mesh_pos, node_type, cells, params,
           is_training):
    n = world_pos.shape[0]
    senders, receivers, ragg = _build_edges(cells, n)
    e = senders.shape[0]
    epad = -(-e // (_NW * _CH)) * (_NW * _CH)
    pad = epad - e
    idt = senders.dtype
    s_p = jnp.concatenate([senders, jnp.zeros((pad,), idt)]).astype(jnp.int32)
    r_p = jnp.concatenate([receivers, jnp.zeros((pad,), idt)]).astype(jnp.int32)
    ragg_p = jnp.concatenate([ragg, jnp.full((pad,), n, idt)]).astype(jnp.int32)
    nrows = -(-(n + 1) // (_NS * 8)) * (_NS * 8)

    pt = jnp.concatenate(
        [world_pos, mesh_pos, jnp.zeros((n, 123), _F32)], axis=1)
    ts, tr = _sc_gather2(n, 128, epad)(pt, pt, s_p, r_p)

    ee = params["edge_encoder"]
    (w1, b1), (w2, b2), (w3, b3) = ee["layers"]
    g, be = ee["ln"]
    w1p, b1p = _fold_first_layer(w1, b1, params["edge_norm_mean"],
                                 params["edge_norm_std"])
    wg = jnp.zeros((128, 128), _F32)
    wg = (wg.at[0:3].set(w1p[0:3]).at[3:5].set(w1p[4:6])
          .at[5].set(w1p[3]).at[6].set(w1p[6]))
    el = _edge_enc_call(epad)(ts, tr, wg, _r2(b1p), w2, _r2(b2), w3, _r2(b3),
                              _r2(g), _r2(be))

    nn_ = params["node_encoder"]
    (w1, b1), (w2, b2), (w3, b3) = nn_["layers"]
    g, be = nn_["ln"]
    w1p, b1p = _fold_first_layer(w1, b1, params["node_norm_mean"],
                                 params["node_norm_std"])
    ew0 = params["blocks"][0]["edge"]["layers"][0][0]
    nl, ps_t, pr_t = _node_enc_call(n)(
        world_pos, prev_world_pos, node_type,
        w1p[0:3], w1p[3:12], _r2(b1p), w2, _r2(b2), w3, _r2(b3),
        _r2(g), _r2(be), ew0[0:128], ew0[128:256])

    zeros_acc = jnp.zeros((nrows, 128), _F32)
    zproj = jnp.zeros((128, 128), _F32)

    nblocks = len(params["blocks"])
    for i, blk in enumerate(params["blocks"]):
        eb = blk["edge"]
        (w1, b1), (w2, b2), (w3, b3) = eb["layers"]
        g, be = eb["ln"]
        gs, gr = _sc_gather2(n, 128, epad)(ps_t, pr_t, s_p, r_p)
        ne, el = _edge_step_call(epad)(
            gs, gr, el, w1[256:384], _r2(b1), w2, _r2(b2), w3, _r2(b3),
            _r2(g), _r2(be))
        a0, a1 = _sc_scatter(nrows, epad)(ne, ragg_p, zeros_acc)
        nb = blk["node"]
        (w1, b1), (w2, b2), (w3, b3) = nb["layers"]
        g, be = nb["ln"]
        if i + 1 < nblocks:
            ewn = params["blocks"][i + 1]["edge"]["layers"][0][0]
            ws, wr = ewn[0:128], ewn[128:256]
        else:
            ws, wr = zproj, zproj
        nl, ps_t, pr_t = _node_step_call(n, nrows)(
            nl, a0, a1, w1[0:128], w1[128:256], _r2(b1), w2, _r2(b2),
            w3, _r2(b3), _r2(g), _r2(be), ws, wr)

    dec = params["decoder"]
    (w1, b1), (w2, b2), (w3, b3) = dec["layers"]
    return _decoder_call(n)(nl, w1, _r2(b1), w2, _r2(b2), w3, _r2(b3))


# speedup vs baseline: 2.4176x; 1.1247x over previous
import functools

import jax
import jax.numpy as jnp
from jax import lax
from jax.experimental import pallas as pl
from jax.experimental.pallas import tpu as pltpu
from jax.experimental.pallas import tpu_sc as plsc

_F32 = jnp.float32
_NC, _NS = 2, 16
_NW = _NC * _NS
_CH = 128
_NB = 1000
_EB = 1920


def _dot(a, b):
    return jnp.dot(a, b, preferred_element_type=_F32)


def _b16(x):
    return x.astype(jnp.bfloat16).astype(_F32)


def _ln(x, g, b):
    mu = jnp.mean(x, axis=-1, keepdims=True)
    xc = x - mu
    var = jnp.mean(xc * xc, axis=-1, keepdims=True)
    return xc / jnp.sqrt(var + 1e-5) * g + b


def _mlp_tail(h, w2, b2, w3, b3, g, be):
    h = jax.nn.relu(_dot(h, w2) + b2)
    x = _dot(h, w3) + b3
    return _ln(x, g, be)


_GCH = 64
_RING = 6


@functools.lru_cache(maxsize=None)
def _sc_gather2(n, width, epad):
    k_per_w = epad // (_NW * _GCH)
    mesh = plsc.VectorSubcoreMesh(core_axis_name="c", subcore_axis_name="s")

    @functools.partial(
        pl.kernel,
        out_type=(jax.ShapeDtypeStruct((epad, width), _F32),
                  jax.ShapeDtypeStruct((epad, width), _F32)),
        mesh=mesh,
        scratch_types=[
            pltpu.VMEM((k_per_w * _GCH,), jnp.int32),
            pltpu.VMEM((k_per_w * _GCH,), jnp.int32),
            pltpu.VMEM((_RING, _GCH, width), _F32),
            pltpu.VMEM((_RING, _GCH, width), _F32),
        ] + [pltpu.SemaphoreType.DMA] * (2 * _RING),
    )
    def k(ta, tb, ia_h, ib_h, ga_h, gb_h, iall_a, iall_b, ra, rb, *sems):
        sg = sems[:_RING]
        sw = sems[_RING:]
        wid = lax.axis_index("s") * _NC + lax.axis_index("c")
        base0 = wid * k_per_w * _GCH
        pltpu.sync_copy(ia_h.at[pl.ds(base0, k_per_w * _GCH)], iall_a)
        pltpu.sync_copy(ib_h.at[pl.ds(base0, k_per_w * _GCH)], iall_b)

        def fire_gather(b, c):
            pltpu.async_copy(ta.at[iall_a.at[pl.ds(c * _GCH, _GCH)]],
                             ra.at[b], sg[b])
            pltpu.async_copy(tb.at[iall_b.at[pl.ds(c * _GCH, _GCH)]],
                             rb.at[b], sg[b])

        def wait_gather(b):
            pltpu.make_async_copy(ta.at[iall_a.at[pl.ds(0, _GCH)]],
                                  ra.at[b], sg[b]).wait()
            pltpu.make_async_copy(tb.at[iall_b.at[pl.ds(0, _GCH)]],
                                  rb.at[b], sg[b]).wait()

        def fire_wb(b, c):
            base = base0 + c * _GCH
            pltpu.async_copy(ra.at[b], ga_h.at[pl.ds(base, _GCH)], sw[b])
            pltpu.async_copy(rb.at[b], gb_h.at[pl.ds(base, _GCH)], sw[b])

        def wait_wb(b):
            pltpu.make_async_copy(ra.at[b], ga_h.at[pl.ds(0, _GCH)],
                                  sw[b]).wait()
            pltpu.make_async_copy(rb.at[b], gb_h.at[pl.ds(0, _GCH)],
                                  sw[b]).wait()

        @pl.loop(0, k_per_w, step=_RING)
        def grp(j):
            for b in range(_RING):
                @pl.when(j > 0)
                def _():
                    wait_wb(b)
                fire_gather(b, j + b)
            for b in range(_RING):
                wait_gather(b)
                fire_wb(b, j + b)

        for b in range(_RING):
            wait_wb(b)

    return k


@functools.lru_cache(maxsize=None)
def _sc_scatter(nrows, epad):
    k_per_w = epad // (_NW * _GCH)
    rpt = nrows // _NS
    mesh = plsc.VectorSubcoreMesh(core_axis_name="c", subcore_axis_name="s")

    @functools.partial(
        pl.kernel,
        out_type=(jax.ShapeDtypeStruct((nrows, 128), _F32),
                  jax.ShapeDtypeStruct((nrows, 128), _F32)),
        mesh=mesh,
        scratch_types=[
            pltpu.VMEM((_RING, _GCH), jnp.int32),
            pltpu.VMEM((_RING, _GCH, 128), _F32),
            pltpu.VMEM_SHARED((nrows, 128), _F32),
        ] + [pltpu.SemaphoreType.DMA] * (2 * _RING),
    )
    def k(ne_h, idx_h, z_h, o0, o1, idx_v, ne_v, acc, *sems):
        sl = sems[:_RING]
        ss = sems[_RING:]
        c = lax.axis_index("c")
        s = lax.axis_index("s")
        wid = s * _NC + c
        r0 = s * rpt
        base0 = wid * k_per_w * _GCH
        pltpu.sync_copy(z_h.at[pl.ds(r0, rpt)], acc.at[pl.ds(r0, rpt)])
        plsc.subcore_barrier()

        def fire_load(b, cidx):
            base = base0 + cidx * _GCH
            pltpu.async_copy(ne_h.at[pl.ds(base, _GCH)], ne_v.at[b], sl[b])
            pltpu.async_copy(idx_h.at[pl.ds(base, _GCH)], idx_v.at[b], sl[b])

        def wait_load(b):
            pltpu.make_async_copy(ne_h.at[pl.ds(0, _GCH)], ne_v.at[b],
                                  sl[b]).wait()
            pltpu.make_async_copy(idx_h.at[pl.ds(0, _GCH)], idx_v.at[b],
                                  sl[b]).wait()

        def fire_scat(b):
            pltpu.async_copy(ne_v.at[b], acc.at[idx_v.at[b]], ss[b],
                             add=True)

        def wait_scat(b):
            pltpu.make_async_copy(ne_v.at[b], acc.at[idx_v.at[b]],
                                  ss[b]).wait()

        @pl.loop(0, k_per_w, step=_RING)
        def grp(j):
            for b in range(_RING):
                @pl.when(j > 0)
                def _():
                    wait_scat(b)
                fire_load(b, j + b)
            for b in range(_RING):
                wait_load(b)
                fire_scat(b)

        for b in range(_RING):
            wait_scat(b)
        plsc.subcore_barrier()

        @pl.when(c == 0)
        def _():
            pltpu.sync_copy(acc.at[pl.ds(r0, rpt)], o0.at[pl.ds(r0, rpt)])

        @pl.when(c == 1)
        def _():
            pltpu.sync_copy(acc.at[pl.ds(r0, rpt)], o1.at[pl.ds(r0, rpt)])

    return k


def _row(i):
    return (i, 0)


def _full(i):
    return (0, 0)


def _w(shape):
    return pl.BlockSpec(shape, _full)


_PARALLEL = pltpu.CompilerParams(dimension_semantics=("parallel",))


@functools.lru_cache(maxsize=None)
def _node_enc_call(n):
    def body(wp, pw, nt, w1v, w1t, b1, w2, b2, w3, b3, g, be, ws, wr,
             nl_ref, ps_ref, pr_ref):
        vel = _b16(wp[...] - pw[...])
        w1vb = _b16(w1v[...])
        w1tb = _b16(w1t[...])
        acc = (vel[:, 0:1] * w1vb[0:1, :] + vel[:, 1:2] * w1vb[1:2, :]
               + vel[:, 2:3] * w1vb[2:3, :]) + b1[...]
        t = nt[...]
        for kk in range(9):
            acc = acc + jnp.where(t == kk, 1.0, 0.0) * w1tb[kk:kk + 1, :]
        h = jax.nn.relu(acc)
        nl = _mlp_tail(h, w2[...], b2[...], w3[...], b3[...], g[...], be[...])
        nl_ref[...] = nl
        ps_ref[...] = _dot(nl, ws[...])
        pr_ref[...] = _dot(nl, wr[...])

    return pl.pallas_call(
        body,
        grid=(n // _NB,),
        in_specs=[
            pl.BlockSpec((_NB, 3), _row), pl.BlockSpec((_NB, 3), _row),
            pl.BlockSpec((_NB, 1), _row),
            _w((3, 128)), _w((9, 128)), _w((1, 128)),
            _w((128, 128)), _w((1, 128)), _w((128, 128)), _w((1, 128)),
            _w((1, 128)), _w((1, 128)), _w((128, 128)), _w((128, 128)),
        ],
        out_specs=[pl.BlockSpec((_NB, 128), _row)] * 3,
        out_shape=[jax.ShapeDtypeStruct((n, 128), _F32)] * 3,
        compiler_params=_PARALLEL,
    )


@functools.lru_cache(maxsize=None)
def _edge_enc_call(epad):
    def body(ts, tr, wg, b1, w2, b2, w3, b3, g, be, el_ref):
        d = ts[...] - tr[...]
        sq = d * d
        col = lax.broadcasted_iota(jnp.int32, d.shape, 1)
        nw = jnp.sqrt(jnp.sum(jnp.where(col < 3, sq, 0.0), axis=1,
                              keepdims=True))
        nm = jnp.sqrt(jnp.sum(jnp.where((col >= 3) & (col < 5), sq, 0.0),
                              axis=1, keepdims=True))
        gf = jnp.where(col == 5, nw, jnp.where(col == 6, nm, d))
        h = jax.nn.relu(_dot(gf, wg[...])
                        + b1[...])
        el_ref[...] = _mlp_tail(h, w2[...], b2[...], w3[...], b3[...],
                                g[...], be[...])

    return pl.pallas_call(
        body,
        grid=(epad // _EB,),
        in_specs=[
            pl.BlockSpec((_EB, 128), _row), pl.BlockSpec((_EB, 128), _row),
            _w((128, 128)), _w((1, 128)),
            _w((128, 128)), _w((1, 128)), _w((128, 128)), _w((1, 128)),
            _w((1, 128)), _w((1, 128)),
        ],
        out_specs=pl.BlockSpec((_EB, 128), _row),
        out_shape=jax.ShapeDtypeStruct((epad, 128), _F32),
        compiler_params=_PARALLEL,
    )


@functools.lru_cache(maxsize=None)
def _edge_step_call(epad):
    def body(gs, gr, el, w1e, b1, w2, b2, w3, b3, g, be, ne_ref, el_ref):
        x0 = el[...]
        h = jax.nn.relu(gs[...] + gr[...]
                        + _dot(x0, w1e[...])
                        + b1[...])
        ne = _mlp_tail(h, w2[...], b2[...], w3[...], b3[...], g[...], be[...])
        ne_ref[...] = ne
        el_ref[...] = x0 + ne

    return pl.pallas_call(
        body,
        grid=(epad // _EB,),
        in_specs=[
            pl.BlockSpec((_EB, 128), _row), pl.BlockSpec((_EB, 128), _row),
            pl.BlockSpec((_EB, 128), _row),
            _w((128, 128)), _w((1, 128)),
            _w((128, 128)), _w((1, 128)), _w((128, 128)), _w((1, 128)),
            _w((1, 128)), _w((1, 128)),
        ],
        out_specs=[pl.BlockSpec((_EB, 128), _row)] * 2,
        out_shape=[jax.ShapeDtypeStruct((epad, 128), _F32)] * 2,
        compiler_params=_PARALLEL,
    )


@functools.lru_cache(maxsize=None)
def _node_step_call(n, nrows):
    def body(nl, a0, a1, wna, wnb, b1, w2, b2, w3, b3, g, be, ws, wr,
             nl_ref, ps_ref, pr_ref):
        x0 = nl[...]
        agg = a0[...] + a1[...]
        h = jax.nn.relu(_dot(x0, wna[...])
                        + _dot(agg, wnb[...])
                        + b1[...])
        nn = _mlp_tail(h, w2[...], b2[...], w3[...], b3[...], g[...], be[...])
        x1 = x0 + nn
        nl_ref[...] = x1
        ps_ref[...] = _dot(x1, ws[...])
        pr_ref[...] = _dot(x1, wr[...])

    return pl.pallas_call(
        body,
        grid=(n // _NB,),
        in_specs=[
            pl.BlockSpec((_NB, 128), _row),
            pl.BlockSpec((_NB, 128), _row), pl.BlockSpec((_NB, 128), _row),
            _w((128, 128)), _w((128, 128)), _w((1, 128)),
            _w((128, 128)), _w((1, 128)), _w((128, 128)), _w((1, 128)),
            _w((1, 128)), _w((1, 128)), _w((128, 128)), _w((128, 128)),
        ],
        out_specs=[pl.BlockSpec((_NB, 128), _row)] * 3,
        out_shape=[jax.ShapeDtypeStruct((n, 128), _F32)] * 3,
        compiler_params=_PARALLEL,
    )


@functools.lru_cache(maxsize=None)
def _decoder_call(n):
    def body(nl, w1, b1, w2, b2, w3, b3, out_ref):
        h = jax.nn.relu(_dot(nl[...], w1[...]) + b1[...])
        h = jax.nn.relu(_dot(h, w2[...]) + b2[...])
        out_ref[...] = _dot(h, w3[...]) + b3[...]

    return pl.pallas_call(
        body,
        grid=(n // _NB,),
        in_specs=[
            pl.BlockSpec((_NB, 128), _row),
            _w((128, 128)), _w((1, 128)), _w((128, 128)), _w((1, 128)),
            _w((128, 3)), _w((1, 3)),
        ],
        out_specs=pl.BlockSpec((_NB, 3), _row),
        out_shape=jax.ShapeDtypeStruct((n, 3), _F32),
        compiler_params=_PARALLEL,
    )


def _build_edges(cells, n):
    e = jnp.concatenate([
        cells[:, 0:2],
        cells[:, 1:3],
        jnp.stack([cells[:, 2], cells[:, 0]], axis=1),
    ], axis=0)
    recv = jnp.min(e, axis=1)
    send = jnp.max(e, axis=1)
    packed = send * n + recv
    ps = jnp.sort(packed)
    is_first = jnp.concatenate([
        jnp.ones((1,), dtype=bool), ps[1:] != ps[:-1]], axis=0)
    s_u = ps // n
    r_u = ps % n
    senders = jnp.concatenate([s_u, r_u], axis=0)
    receivers = jnp.concatenate([r_u, s_u], axis=0)
    valid = jnp.concatenate([is_first, is_first], axis=0)
    ragg = jnp.where(valid, receivers, n)
    return senders, receivers, ragg


def _fold_first_layer(w1, b1, mean, std):
    w1p = w1 / std[:, None]
    b1p = b1 - (mean / std) @ w1
    return w1p, b1p


def _r2(b):
    return b.reshape(1, -1)


def kernel(world_pos, prev_world_pos, mesh_pos, node_type, cells, params,
           is_training):
    n = world_pos.shape[0]
    senders, receivers, ragg = _build_edges(cells, n)
    e = senders.shape[0]
    quantum = _NW * _GCH * _RING
    epad = -(-e // quantum) * quantum
    pad = epad - e
    idt = senders.dtype
    s_p = jnp.concatenate([senders, jnp.zeros((pad,), idt)]).astype(jnp.int32)
    r_p = jnp.concatenate([receivers, jnp.zeros((pad,), idt)]).astype(jnp.int32)
    ragg_p = jnp.concatenate([ragg, jnp.full((pad,), n, idt)]).astype(jnp.int32)
    nrows = -(-(n + 1) // (_NS * 8)) * (_NS * 8)

    pt = jnp.concatenate(
        [world_pos, mesh_pos, jnp.zeros((n, 123), _F32)], axis=1)
    ts, tr = _sc_gather2(n, 128, epad)(pt, pt, s_p, r_p)

    ee = params["edge_encoder"]
    (w1, b1), (w2, b2), (w3, b3) = ee["layers"]
    g, be = ee["ln"]
    w1p, b1p = _fold_first_layer(w1, b1, params["edge_norm_mean"],
                                 params["edge_norm_std"])
    wg = jnp.zeros((128, 128), _F32)
    wg = (wg.at[0:3].set(w1p[0:3]).at[3:5].set(w1p[4:6])
          .at[5].set(w1p[3]).at[6].set(w1p[6]))
    el = _edge_enc_call(epad)(ts, tr, wg, _r2(b1p), w2, _r2(b2), w3, _r2(b3),
                              _r2(g), _r2(be))

    nn_ = params["node_encoder"]
    (w1, b1), (w2, b2), (w3, b3) = nn_["layers"]
    g, be = nn_["ln"]
    w1p, b1p = _fold_first_layer(w1, b1, params["node_norm_mean"],
                                 params["node_norm_std"])
    ew0 = params["blocks"][0]["edge"]["layers"][0][0]
    nl, ps_t, pr_t = _node_enc_call(n)(
        world_pos, prev_world_pos, node_type,
        w1p[0:3], w1p[3:12], _r2(b1p), w2, _r2(b2), w3, _r2(b3),
        _r2(g), _r2(be), ew0[0:128], ew0[128:256])

    zeros_acc = jnp.zeros((nrows, 128), _F32)
    zproj = jnp.zeros((128, 128), _F32)

    nblocks = len(params["blocks"])
    for i, blk in enumerate(params["blocks"]):
        eb = blk["edge"]
        (w1, b1), (w2, b2), (w3, b3) = eb["layers"]
        g, be = eb["ln"]
        gs, gr = _sc_gather2(n, 128, epad)(ps_t, pr_t, s_p, r_p)
        ne, el = _edge_step_call(epad)(
            gs, gr, el, w1[256:384], _r2(b1), w2, _r2(b2), w3, _r2(b3),
            _r2(g), _r2(be))
        a0, a1 = _sc_scatter(nrows, epad)(ne, ragg_p, zeros_acc)
        nb = blk["node"]
        (w1, b1), (w2, b2), (w3, b3) = nb["layers"]
        g, be = nb["ln"]
        if i + 1 < nblocks:
            ewn = params["blocks"][i + 1]["edge"]["layers"][0][0]
            ws, wr = ewn[0:128], ewn[128:256]
        else:
            ws, wr = zproj, zproj
        nl, ps_t, pr_t = _node_step_call(n, nrows)(
            nl, a0, a1, w1[0:128], w1[128:256], _r2(b1), w2, _r2(b2),
            w3, _r2(b3), _r2(g), _r2(be), ws, wr)

    dec = params["decoder"]
    (w1, b1), (w2, b2), (w3, b3) = dec["layers"]
    return _decoder_call(n)(nl, w1, _r2(b1), w2, _r2(b2), w3, _r2(b3))


# speedup vs baseline: 3.3664x; 1.3924x over previous
import functools

import jax
import jax.numpy as jnp
from jax import lax
from jax.experimental import pallas as pl
from jax.experimental.pallas import tpu as pltpu
from jax.experimental.pallas import tpu_sc as plsc

_F32 = jnp.float32
_NC, _NS = 2, 16
_NW = _NC * _NS
_CH = 128
_NB = 1000
_EB = 1920


def _dot(a, b):
    return jnp.dot(a, b, preferred_element_type=_F32)


def _b16(x):
    return x.astype(jnp.bfloat16).astype(_F32)


def _ln(x, g, b):
    mu = jnp.mean(x, axis=-1, keepdims=True)
    xc = x - mu
    var = jnp.mean(xc * xc, axis=-1, keepdims=True)
    return xc / jnp.sqrt(var + 1e-5) * g + b


def _mlp_tail(h, w2, b2, w3, b3, g, be):
    h = jax.nn.relu(_dot(h, w2) + b2)
    x = _dot(h, w3) + b3
    return _ln(x, g, be)


_GCH = 64
_RING = 6


@functools.lru_cache(maxsize=None)
def _sc_gather2(n, width, epad):
    k_per_w = epad // (_NW * _GCH)
    mesh = plsc.VectorSubcoreMesh(core_axis_name="c", subcore_axis_name="s")

    @functools.partial(
        pl.kernel,
        out_type=(jax.ShapeDtypeStruct((epad, width), _F32),
                  jax.ShapeDtypeStruct((epad, width), _F32)),
        mesh=mesh,
        scratch_types=[
            pltpu.VMEM((k_per_w * _GCH,), jnp.int32),
            pltpu.VMEM((k_per_w * _GCH,), jnp.int32),
            pltpu.VMEM((_RING, _GCH, width), _F32),
            pltpu.VMEM((_RING, _GCH, width), _F32),
        ] + [pltpu.SemaphoreType.DMA] * (2 * _RING),
    )
    def k(ta, tb, ia_h, ib_h, ga_h, gb_h, iall_a, iall_b, ra, rb, *sems):
        sg = sems[:_RING]
        sw = sems[_RING:]
        wid = lax.axis_index("s") * _NC + lax.axis_index("c")
        base0 = wid * k_per_w * _GCH
        pltpu.sync_copy(ia_h.at[pl.ds(base0, k_per_w * _GCH)], iall_a)
        pltpu.sync_copy(ib_h.at[pl.ds(base0, k_per_w * _GCH)], iall_b)

        def fire_gather(b, c):
            pltpu.async_copy(ta.at[iall_a.at[pl.ds(c * _GCH, _GCH)]],
                             ra.at[b], sg[b])
            pltpu.async_copy(tb.at[iall_b.at[pl.ds(c * _GCH, _GCH)]],
                             rb.at[b], sg[b])

        def wait_gather(b):
            pltpu.make_async_copy(ta.at[iall_a.at[pl.ds(0, _GCH)]],
                                  ra.at[b], sg[b]).wait()
            pltpu.make_async_copy(tb.at[iall_b.at[pl.ds(0, _GCH)]],
                                  rb.at[b], sg[b]).wait()

        def fire_wb(b, c):
            base = base0 + c * _GCH
            pltpu.async_copy(ra.at[b], ga_h.at[pl.ds(base, _GCH)], sw[b])
            pltpu.async_copy(rb.at[b], gb_h.at[pl.ds(base, _GCH)], sw[b])

        def wait_wb(b):
            pltpu.make_async_copy(ra.at[b], ga_h.at[pl.ds(0, _GCH)],
                                  sw[b]).wait()
            pltpu.make_async_copy(rb.at[b], gb_h.at[pl.ds(0, _GCH)],
                                  sw[b]).wait()

        @pl.loop(0, k_per_w, step=_RING)
        def grp(j):
            for b in range(_RING):
                @pl.when(j > 0)
                def _():
                    wait_wb(b)
                fire_gather(b, j + b)
            for b in range(_RING):
                wait_gather(b)
                fire_wb(b, j + b)

        for b in range(_RING):
            wait_wb(b)

    return k


@functools.lru_cache(maxsize=None)
def _sc_scatter(nrows, hpad):
    k_per_w = hpad // (_NW * _GCH)
    rpt = nrows // _NS
    mesh = plsc.VectorSubcoreMesh(core_axis_name="c", subcore_axis_name="s")

    @functools.partial(
        pl.kernel,
        out_type=(jax.ShapeDtypeStruct((nrows, 128), _F32),
                  jax.ShapeDtypeStruct((nrows, 128), _F32)),
        mesh=mesh,
        scratch_types=[
            pltpu.VMEM((_RING, _GCH), jnp.int32),
            pltpu.VMEM((_RING, _GCH, 128), _F32),
            pltpu.VMEM_SHARED((nrows, 128), _F32),
        ] + [pltpu.SemaphoreType.DMA] * (2 * _RING),
    )
    def k(ne1_h, ne2_h, idx1_h, idx2_h, z_h, o0, o1, idx_v, ne_v, acc,
          *sems):
        sl = sems[:_RING]
        ss = sems[_RING:]
        c = lax.axis_index("c")
        s = lax.axis_index("s")
        wid = s * _NC + c
        r0 = s * rpt
        base0 = wid * k_per_w * _GCH
        pltpu.sync_copy(z_h.at[pl.ds(r0, rpt)], acc.at[pl.ds(r0, rpt)])
        plsc.subcore_barrier()

        for ne_h, idx_h in ((ne1_h, idx1_h), (ne2_h, idx2_h)):
            def fire_load(b, cidx, ne_h=ne_h, idx_h=idx_h):
                base = base0 + cidx * _GCH
                pltpu.async_copy(ne_h.at[pl.ds(base, _GCH)], ne_v.at[b],
                                 sl[b])
                pltpu.async_copy(idx_h.at[pl.ds(base, _GCH)], idx_v.at[b],
                                 sl[b])

            def wait_load(b, ne_h=ne_h, idx_h=idx_h):
                pltpu.make_async_copy(ne_h.at[pl.ds(0, _GCH)], ne_v.at[b],
                                      sl[b]).wait()
                pltpu.make_async_copy(idx_h.at[pl.ds(0, _GCH)], idx_v.at[b],
                                      sl[b]).wait()

            def fire_scat(b):
                pltpu.async_copy(ne_v.at[b], acc.at[idx_v.at[b]], ss[b],
                                 add=True)

            def wait_scat(b):
                pltpu.make_async_copy(ne_v.at[b], acc.at[idx_v.at[b]],
                                      ss[b]).wait()

            @pl.loop(0, k_per_w, step=_RING)
            def grp(j, fire_load=fire_load, wait_load=wait_load,
                    fire_scat=fire_scat, wait_scat=wait_scat):
                for b in range(_RING):
                    @pl.when(j > 0)
                    def _():
                        wait_scat(b)
                    fire_load(b, j + b)
                for b in range(_RING):
                    wait_load(b)
                    fire_scat(b)

            for b in range(_RING):
                wait_scat(b)
        plsc.subcore_barrier()

        @pl.when(c == 0)
        def _():
            pltpu.sync_copy(acc.at[pl.ds(r0, rpt)], o0.at[pl.ds(r0, rpt)])

        @pl.when(c == 1)
        def _():
            pltpu.sync_copy(acc.at[pl.ds(r0, rpt)], o1.at[pl.ds(r0, rpt)])

    return k


def _row(i):
    return (i, 0)


def _full(i):
    return (0, 0)


def _w(shape):
    return pl.BlockSpec(shape, _full)


_PARALLEL = pltpu.CompilerParams(dimension_semantics=("parallel",))


@functools.lru_cache(maxsize=None)
def _node_enc_call(n):
    def body(wp, pw, nt, w1v, w1t, b1, w2, b2, w3, b3, g, be, nl_ref):
        vel = _b16(wp[...] - pw[...])
        w1vb = _b16(w1v[...])
        w1tb = _b16(w1t[...])
        acc = (vel[:, 0:1] * w1vb[0:1, :] + vel[:, 1:2] * w1vb[1:2, :]
               + vel[:, 2:3] * w1vb[2:3, :]) + b1[...]
        t = nt[...]
        for kk in range(9):
            acc = acc + jnp.where(t == kk, 1.0, 0.0) * w1tb[kk:kk + 1, :]
        h = jax.nn.relu(acc)
        nl_ref[...] = _mlp_tail(h, w2[...], b2[...], w3[...], b3[...],
                                g[...], be[...])

    return pl.pallas_call(
        body,
        grid=(n // _NB,),
        in_specs=[
            pl.BlockSpec((_NB, 3), _row), pl.BlockSpec((_NB, 3), _row),
            pl.BlockSpec((_NB, 1), _row),
            _w((3, 128)), _w((9, 128)), _w((1, 128)),
            _w((128, 128)), _w((1, 128)), _w((128, 128)), _w((1, 128)),
            _w((1, 128)), _w((1, 128)),
        ],
        out_specs=pl.BlockSpec((_NB, 128), _row),
        out_shape=jax.ShapeDtypeStruct((n, 128), _F32),
        compiler_params=_PARALLEL,
    )


@functools.lru_cache(maxsize=None)
def _edge_enc_call(hpad):
    def body(ts, tr, wg, b1, w2, b2, w3, b3, g, be, el1_ref, el2_ref):
        d = ts[...] - tr[...]
        sq = d * d
        col = lax.broadcasted_iota(jnp.int32, d.shape, 1)
        nw = jnp.sqrt(jnp.sum(jnp.where(col < 3, sq, 0.0), axis=1,
                              keepdims=True))
        nm = jnp.sqrt(jnp.sum(jnp.where((col >= 3) & (col < 5), sq, 0.0),
                              axis=1, keepdims=True))
        gf1 = jnp.where(col == 5, nw, jnp.where(col == 6, nm, d))
        gf2 = jnp.where(col == 5, nw, jnp.where(col == 6, nm, -d))
        h1 = jax.nn.relu(_dot(gf1, wg[...]) + b1[...])
        h2 = jax.nn.relu(_dot(gf2, wg[...]) + b1[...])
        el1_ref[...] = _mlp_tail(h1, w2[...], b2[...], w3[...], b3[...],
                                 g[...], be[...])
        el2_ref[...] = _mlp_tail(h2, w2[...], b2[...], w3[...], b3[...],
                                 g[...], be[...])

    return pl.pallas_call(
        body,
        grid=(hpad // _EB,),
        in_specs=[
            pl.BlockSpec((_EB, 128), _row), pl.BlockSpec((_EB, 128), _row),
            _w((128, 128)), _w((1, 128)),
            _w((128, 128)), _w((1, 128)), _w((128, 128)), _w((1, 128)),
            _w((1, 128)), _w((1, 128)),
        ],
        out_specs=[pl.BlockSpec((_EB, 128), _row)] * 2,
        out_shape=[jax.ShapeDtypeStruct((hpad, 128), _F32)] * 2,
        compiler_params=_PARALLEL,
    )


@functools.lru_cache(maxsize=None)
def _edge_step_call(hpad):
    def body(g1, g2, el1, el2, w1s, w1r, w1e, b1, w2, b2, w3, b3, g, be,
             ne1_ref, ne2_ref, el1_ref, el2_ref):
        gs1 = _dot(g1[...], w1s[...])
        gr1 = _dot(g2[...], w1r[...])
        gs2 = _dot(g2[...], w1s[...])
        gr2 = _dot(g1[...], w1r[...])
        x1 = el1[...]
        x2 = el2[...]
        h1 = jax.nn.relu(gs1 + gr1 + _dot(x1, w1e[...]) + b1[...])
        h2 = jax.nn.relu(gs2 + gr2 + _dot(x2, w1e[...]) + b1[...])
        ne1 = _mlp_tail(h1, w2[...], b2[...], w3[...], b3[...], g[...],
                        be[...])
        ne2 = _mlp_tail(h2, w2[...], b2[...], w3[...], b3[...], g[...],
                        be[...])
        ne1_ref[...] = ne1
        ne2_ref[...] = ne2
        el1_ref[...] = x1 + ne1
        el2_ref[...] = x2 + ne2

    return pl.pallas_call(
        body,
        grid=(hpad // _EB,),
        in_specs=[
            pl.BlockSpec((_EB, 128), _row), pl.BlockSpec((_EB, 128), _row),
            pl.BlockSpec((_EB, 128), _row), pl.BlockSpec((_EB, 128), _row),
            _w((128, 128)), _w((128, 128)), _w((128, 128)), _w((1, 128)),
            _w((128, 128)), _w((1, 128)), _w((128, 128)), _w((1, 128)),
            _w((1, 128)), _w((1, 128)),
        ],
        out_specs=[pl.BlockSpec((_EB, 128), _row)] * 4,
        out_shape=[jax.ShapeDtypeStruct((hpad, 128), _F32)] * 4,
        compiler_params=_PARALLEL,
    )


@functools.lru_cache(maxsize=None)
def _node_step_call(n, nrows):
    def body(nl, a0, a1, wna, wnb, b1, w2, b2, w3, b3, g, be, nl_ref):
        x0 = nl[...]
        agg = a0[...] + a1[...]
        h = jax.nn.relu(_dot(x0, wna[...])
                        + _dot(agg, wnb[...])
                        + b1[...])
        nn = _mlp_tail(h, w2[...], b2[...], w3[...], b3[...], g[...], be[...])
        nl_ref[...] = x0 + nn

    return pl.pallas_call(
        body,
        grid=(n // _NB,),
        in_specs=[
            pl.BlockSpec((_NB, 128), _row),
            pl.BlockSpec((_NB, 128), _row), pl.BlockSpec((_NB, 128), _row),
            _w((128, 128)), _w((128, 128)), _w((1, 128)),
            _w((128, 128)), _w((1, 128)), _w((128, 128)), _w((1, 128)),
            _w((1, 128)), _w((1, 128)),
        ],
        out_specs=pl.BlockSpec((_NB, 128), _row),
        out_shape=jax.ShapeDtypeStruct((n, 128), _F32),
        compiler_params=_PARALLEL,
    )


@functools.lru_cache(maxsize=None)
def _decoder_call(n):
    def body(nl, w1, b1, w2, b2, w3, b3, out_ref):
        h = jax.nn.relu(_dot(nl[...], w1[...]) + b1[...])
        h = jax.nn.relu(_dot(h, w2[...]) + b2[...])
        out_ref[...] = _dot(h, w3[...]) + b3[...]

    return pl.pallas_call(
        body,
        grid=(n // _NB,),
        in_specs=[
            pl.BlockSpec((_NB, 128), _row),
            _w((128, 128)), _w((1, 128)), _w((128, 128)), _w((1, 128)),
            _w((128, 3)), _w((1, 3)),
        ],
        out_specs=pl.BlockSpec((_NB, 3), _row),
        out_shape=jax.ShapeDtypeStruct((n, 3), _F32),
        compiler_params=_PARALLEL,
    )


def _build_edges(cells, n):
    e = jnp.concatenate([
        cells[:, 0:2],
        cells[:, 1:3],
        jnp.stack([cells[:, 2], cells[:, 0]], axis=1),
    ], axis=0)
    recv = jnp.min(e, axis=1)
    send = jnp.max(e, axis=1)
    packed = send * n + recv
    ps = jnp.sort(packed)
    is_first = jnp.concatenate([
        jnp.ones((1,), dtype=bool), ps[1:] != ps[:-1]], axis=0)
    s_u = ps // n
    r_u = ps % n
    return s_u, r_u, is_first


def _fold_first_layer(w1, b1, mean, std):
    w1p = w1 / std[:, None]
    b1p = b1 - (mean / std) @ w1
    return w1p, b1p


def _r2(b):
    return b.reshape(1, -1)


def kernel(world_pos, prev_world_pos, mesh_pos, node_type, cells, params,
           is_training):
    n = world_pos.shape[0]
    s_u, r_u, is_first = _build_edges(cells, n)
    h = s_u.shape[0]
    quantum = _NW * _GCH * _RING
    hpad = -(-h // quantum) * quantum
    pad = hpad - h
    idt = s_u.dtype
    s_p = jnp.concatenate([s_u, jnp.zeros((pad,), idt)]).astype(jnp.int32)
    r_p = jnp.concatenate([r_u, jnp.zeros((pad,), idt)]).astype(jnp.int32)
    ragg1 = jnp.concatenate([jnp.where(is_first, r_u, n),
                             jnp.full((pad,), n, idt)]).astype(jnp.int32)
    ragg2 = jnp.concatenate([jnp.where(is_first, s_u, n),
                             jnp.full((pad,), n, idt)]).astype(jnp.int32)
    nrows = -(-(n + 1) // (_NS * 8)) * (_NS * 8)

    pt = jnp.concatenate(
        [world_pos, mesh_pos, jnp.zeros((n, 123), _F32)], axis=1)
    ts, tr = _sc_gather2(n, 128, hpad)(pt, pt, s_p, r_p)

    ee = params["edge_encoder"]
    (w1, b1), (w2, b2), (w3, b3) = ee["layers"]
    g, be = ee["ln"]
    w1p, b1p = _fold_first_layer(w1, b1, params["edge_norm_mean"],
                                 params["edge_norm_std"])
    wg = jnp.zeros((128, 128), _F32)
    wg = (wg.at[0:3].set(w1p[0:3]).at[3:5].set(w1p[4:6])
          .at[5].set(w1p[3]).at[6].set(w1p[6]))
    el1, el2 = _edge_enc_call(hpad)(ts, tr, wg, _r2(b1p), w2, _r2(b2), w3,
                                    _r2(b3), _r2(g), _r2(be))

    nn_ = params["node_encoder"]
    (w1, b1), (w2, b2), (w3, b3) = nn_["layers"]
    g, be = nn_["ln"]
    w1p, b1p = _fold_first_layer(w1, b1, params["node_norm_mean"],
                                 params["node_norm_std"])
    nl = _node_enc_call(n)(
        world_pos, prev_world_pos, node_type,
        w1p[0:3], w1p[3:12], _r2(b1p), w2, _r2(b2), w3, _r2(b3),
        _r2(g), _r2(be))

    zeros_acc = jnp.zeros((nrows, 128), _F32)

    for blk in params["blocks"]:
        eb = blk["edge"]
        (w1, b1), (w2, b2), (w3, b3) = eb["layers"]
        g, be = eb["ln"]
        g1, g2 = _sc_gather2(n, 128, hpad)(nl, nl, s_p, r_p)
        ne1, ne2, el1, el2 = _edge_step_call(hpad)(
            g1, g2, el1, el2, w1[0:128], w1[128:256], w1[256:384],
            _r2(b1), w2, _r2(b2), w3, _r2(b3), _r2(g), _r2(be))
        a0, a1 = _sc_scatter(nrows, hpad)(ne1, ne2, ragg1, ragg2, zeros_acc)
        nb = blk["node"]
        (w1, b1), (w2, b2), (w3, b3) = nb["layers"]
        g, be = nb["ln"]
        nl = _node_step_call(n, nrows)(
            nl, a0, a1, w1[0:128], w1[128:256], _r2(b1), w2, _r2(b2),
            w3, _r2(b3), _r2(g), _r2(be))

    dec = params["decoder"]
    (w1, b1), (w2, b2), (w3, b3) = dec["layers"]
    return _decoder_call(n)(nl, w1, _r2(b1), w2, _r2(b2), w3, _r2(b3))
